# baseline (speedup 1.0000x reference)
import functools

import jax
import jax.numpy as jnp
from jax import lax
from jax.experimental import pallas as pl
from jax.experimental.pallas import tpu as pltpu
from jax.experimental.pallas import tpu_sc as plsc

H = 64
NB = 32
F32 = jnp.float32


def _mm(a, b):
    return jnp.dot(a, b, preferred_element_type=F32)


def tc_encode(vnf, cnf, w1v, b1v, w2v, b2v, w1c, b1c, w2c, b2c):
    def body(vnf_r, cnf_r, w1v_r, b1v_r, w2v_r, b2v_r, w1c_r, b1c_r, w2c_r,
             b2c_r, xv0_r, xc0_r):
        xv0_r[...] = _mm(jax.nn.relu(_mm(vnf_r[...], w1v_r[...]) + b1v_r[...]),
                         w2v_r[...]) + b2v_r[...]
        xc0_r[...] = _mm(jax.nn.relu(_mm(cnf_r[...], w1c_r[...]) + b1c_r[...]),
                         w2c_r[...]) + b2c_r[...]

    nv, nc = vnf.shape[0], cnf.shape[0]
    return pl.pallas_call(
        body,
        out_shape=(jax.ShapeDtypeStruct((nv, H), F32),
                   jax.ShapeDtypeStruct((nc, H), F32)),
    )(vnf, cnf, w1v, b1v, w2v, b2v, w1c, b1c, w2c, b2c)


def tc_layer_prep(xv, xc, a1w, a1b, a2w, a2b, w1a, w1b, w1c):
    def body(xv_r, xc_r, a1w_r, a1b_r, a2w_r, a2b_r, w1a_r, w1b_r, w1c_r,
             a_r, acon_r, bvar_r):
        h = jax.nn.relu(_mm(xv_r[...], a1w_r[...]) + a1b_r[...])
        a = jax.nn.sigmoid(_mm(h, a2w_r[...]) + a2b_r[...])
        a_r[...] = a
        acon_r[...] = _mm(xc_r[...], w1a_r[...])
        bvar_r[...] = _mm(xv_r[...], w1b_r[...]) + a * w1c_r[...]

    nv, nc = xv.shape[0], xc.shape[0]
    return pl.pallas_call(
        body,
        out_shape=(jax.ShapeDtypeStruct((nv, 1), F32),
                   jax.ShapeDtypeStruct((nc, H), F32),
                   jax.ShapeDtypeStruct((nv, H), F32)),
    )(xv, xc, a1w, a1b, a2w, a2b, w1a, w1b, w1c)


def tc_edge_stats(efv, efc, ws):
    E = efv.shape[0]
    EB = 8000
    grid = E // EB

    def body(efv_r, efc_r, *rest):
        wrefs = rest[:16]
        out_r = rest[16]
        i = pl.program_id(0)
        accs = []
        for k in range(4):
            ef = efv_r[...] if k < 2 else efc_r[...]
            w1, b1, w2, b2 = (wrefs[4 * k][...], wrefs[4 * k + 1][...],
                              wrefs[4 * k + 2][...], wrefs[4 * k + 3][...])
            t1 = jax.nn.relu(ef * w1 + b1)
            g = jax.nn.relu(_mm(t1, w2) + b2)
            accs.append(jnp.sum(g, axis=0, keepdims=True))
            accs.append(jnp.sum(g * g, axis=0, keepdims=True))
        upd = jnp.concatenate(accs, axis=0)

        @pl.when(i == 0)
        def _():
            out_r[...] = upd

        @pl.when(i > 0)
        def _():
            out_r[...] = out_r[...] + upd

    flat_ws = [w for enc in ws for w in enc]
    in_specs = [pl.BlockSpec((EB, 1), lambda i: (i, 0)),
                pl.BlockSpec((EB, 1), lambda i: (i, 0))]
    for enc in ws:
        in_specs += [pl.BlockSpec(w.shape, lambda i: (0,) * w.ndim)
                     for w in enc]
    return pl.pallas_call(
        body,
        grid=(grid,),
        in_specs=in_specs,
        out_specs=pl.BlockSpec((8, H), lambda i: (0, 0)),
        out_shape=jax.ShapeDtypeStruct((8, H), F32),
    )(efv, efc, *flat_ws)


def tc_err(tmp_parts, rhs, index2d, w1, b1, w2, b2, bng, bnb):
    nc = rhs.shape[0]

    def body(tp_r, rhs_r, idx_r, w1_r, b1_r, w2_r, b2_r, g_r, be_r, err_r):
        tmp = jnp.sum(tp_r[...], axis=0).reshape(nc, 1)
        u = tmp - rhs_r[...]
        h = jax.nn.relu(u * w1_r[...] + b1_r[...])
        h = jax.nn.relu(_mm(h, w2_r[...]) + b2_r[...])
        mu = jnp.mean(h, axis=0, keepdims=True)
        var = jnp.mean(h * h, axis=0, keepdims=True) - mu * mu
        hbn = (h - mu) * (g_r[...] * jax.lax.rsqrt(var + 1e-5)) + be_r[...]
        m = jnp.max(hbn, axis=0, keepdims=True)
        ex = jnp.exp(hbn - m)
        onehot = (idx_r[...] == jax.lax.broadcasted_iota(
            jnp.int32, (1, NB), 1)).astype(F32)
        gs = jax.lax.dot_general(onehot, ex, (((0,), (0,)), ((), ())),
                                 preferred_element_type=F32)
        denom = _mm(onehot, gs)
        err_r[...] = ex / (denom + 1e-16)

    return pl.pallas_call(
        body,
        out_shape=jax.ShapeDtypeStruct((nc, H), F32),
    )(tmp_parts, rhs, index2d, w1, b1, w2, b2, bng, bnb)


def tc_edge_main(ag, bg, ef, ew1, eb1, ew2, eb2, wd, c0, w2n, b2n):
    E = ag.shape[0]
    EB = 8000
    grid = E // EB

    def body(ag_r, bg_r, ef_r, ew1_r, eb1_r, ew2_r, eb2_r, wd_r, c0_r, w2n_r,
             b2n_r, h2_r, st_r):
        i = pl.program_id(0)
        t1 = jax.nn.relu(ef_r[...] * ew1_r[...] + eb1_r[...])
        g = jax.nn.relu(_mm(t1, ew2_r[...]) + eb2_r[...])
        h1 = jax.nn.relu(ag_r[...] + bg_r[...] + _mm(g, wd_r[...]) + c0_r[...])
        h2 = jax.nn.relu(_mm(h1, w2n_r[...]) + b2n_r[...])
        h2_r[...] = h2
        upd = jnp.concatenate([jnp.sum(h2, axis=0, keepdims=True),
                               jnp.sum(h2 * h2, axis=0, keepdims=True)],
                              axis=0)

        @pl.when(i == 0)
        def _():
            st_r[...] = upd

        @pl.when(i > 0)
        def _():
            st_r[...] = st_r[...] + upd

    def c(shape):
        return pl.BlockSpec(shape, lambda i: (0,) * len(shape))

    return pl.pallas_call(
        body,
        grid=(grid,),
        in_specs=[pl.BlockSpec((EB, H), lambda i: (i, 0)),
                  pl.BlockSpec((EB, H), lambda i: (i, 0)),
                  pl.BlockSpec((EB, 1), lambda i: (i, 0)),
                  c((1, H)), c((1, H)), c((H, H)), c((1, H)), c((H, H)),
                  c((1, H)), c((H, H)), c((1, H))],
        out_specs=(pl.BlockSpec((EB, H), lambda i: (i, 0)),
                   pl.BlockSpec((2, H), lambda i: (0, 0))),
        out_shape=(jax.ShapeDtypeStruct((E, H), F32),
                   jax.ShapeDtypeStruct((2, H), F32)),
    )(ag, bg, ef, ew1, eb1, ew2, eb2, wd, c0, w2n, b2n)


def tc_node_varN(s0, s1, cnt2d, stats, bng, bnb, err, xv, w2a, w2b, w2c,
                 e_count):
    nc, nv = s0.shape[0], xv.shape[0]

    def body(s0_r, s1_r, cnt_r, st_r, g_r, be_r, err_r, xv_r, w2a_r, w2b_r,
             w2c_r, xc_r, p_r, q_r):
        cnt = jnp.sum(cnt_r[...], axis=0).reshape(nc, 1)
        mu = st_r[0:1, :] / e_count
        var = st_r[1:2, :] / e_count - mu * mu
        s = g_r[...] * jax.lax.rsqrt(var + 1e-5)
        t = be_r[...] - mu * s
        seg = (s0_r[...] + s1_r[...]) * s + t * cnt
        xc = jax.nn.relu(seg / jnp.maximum(cnt, 1.0))
        xc_r[...] = xc
        p_r[...] = _mm(xc, w2b_r[...]) + _mm(err_r[...], w2c_r[...])
        q_r[...] = _mm(xv_r[...], w2a_r[...])

    return pl.pallas_call(
        body,
        out_shape=(jax.ShapeDtypeStruct((nc, H), F32),
                   jax.ShapeDtypeStruct((nc, H), F32),
                   jax.ShapeDtypeStruct((nv, H), F32)),
    )(s0, s1, cnt2d, stats, bng, bnb, err, xv, w2a, w2b, w2c)


def tc_node_conN(s0, s1, cnt2d, stats, bng, bnb, e_count):
    nv = s0.shape[0]

    def body(s0_r, s1_r, cnt_r, st_r, g_r, be_r, xv_r):
        cnt = jnp.sum(cnt_r[...], axis=0).reshape(nv, 1)
        mu = st_r[0:1, :] / e_count
        var = st_r[1:2, :] / e_count - mu * mu
        s = g_r[...] * jax.lax.rsqrt(var + 1e-5)
        t = be_r[...] - mu * s
        seg = (s0_r[...] + s1_r[...]) * s + t * cnt
        xv_r[...] = jax.nn.relu(seg / jnp.maximum(cnt, 1.0))

    return pl.pallas_call(
        body,
        out_shape=jax.ShapeDtypeStruct((nv, H), F32),
    )(s0, s1, cnt2d, stats, bng, bnb)


def tc_final(xv0, xv1, xv2, w1, b1, w2, b2, w3, b3, w4, b4):
    nv = xv0.shape[0]

    def body(x0_r, x1_r, x2_r, w1_r, b1_r, w2_r, b2_r, w3_r, b3_r, w4_r,
             b4_r, o_r):
        x = jnp.concatenate([x0_r[...], x1_r[...], x2_r[...]], axis=-1)
        x = jax.nn.relu(_mm(x, w1_r[...]) + b1_r[...])
        x = jax.nn.relu(_mm(x, w2_r[...]) + b2_r[...])
        x = jax.nn.relu(_mm(x, w3_r[...]) + b3_r[...])
        o_r[...] = _mm(x, w4_r[...]) + b4_r[...]

    return pl.pallas_call(
        body,
        out_shape=jax.ShapeDtypeStruct((nv, 1), F32),
    )(xv0, xv1, xv2, w1, b1, w2, b2, w3, b3, w4, b4)


_MESH = plsc.VectorSubcoreMesh(core_axis_name="c", subcore_axis_name="s")
NWORK = 32
CH = 128


def _wid():
    return lax.axis_index("s") * 2 + lax.axis_index("c")


def _load_idx_rows(idx_hbm, idx2d, base, nfull, last_off, sem):
    for j in range(nfull):
        pltpu.make_async_copy(idx_hbm.at[pl.ds(base + j * CH, CH)],
                              idx2d.at[j], sem).start()
    pltpu.make_async_copy(idx_hbm.at[pl.ds(base + last_off, CH)],
                          idx2d.at[nfull], sem).start()
    for j in range(nfull + 1):
        pltpu.make_async_copy(idx_hbm.at[pl.ds(base, CH)],
                              idx2d.at[0], sem).wait()


def sc_gather_pair(t1, i1, t2, i2, tmp_args=None, do_cnt=False):
    E = i1.shape[0]
    n1, n2 = t1.shape[0], t2.shape[0]
    ept = E // NWORK
    nfull = ept // CH
    rem = ept - nfull * CH
    last_off = ept - CH
    nch = nfull + (1 if rem else 0)

    n1s, n2s = n1 // 16, n2 // 16
    do_tmp = tmp_args is not None
    stage2 = not (do_tmp or do_cnt)

    out_type = [jax.ShapeDtypeStruct((E, H), F32),
                jax.ShapeDtypeStruct((E, H), F32)]
    scratch = [
        pltpu.VMEM((nch, CH), jnp.int32),
        pltpu.VMEM((nch, CH), jnp.int32),
        pltpu.VMEM((2, CH, H), F32),
        pltpu.VMEM((2, CH, H), F32),
        pltpu.VMEM_SHARED((n1, H), F32),
        pltpu.VMEM_SHARED((n2 if stage2 else 16, H), F32),
        pltpu.SemaphoreType.DMA,
        pltpu.SemaphoreType.DMA,
        pltpu.SemaphoreType.DMA,
        pltpu.SemaphoreType.DMA,
        pltpu.SemaphoreType.DMA,
        pltpu.SemaphoreType.DMA,
    ]
    extra_in = []
    if do_tmp:
        out_type.append(jax.ShapeDtypeStruct((NWORK, n1), F32))
        scratch += [pltpu.VMEM((n2,), F32), pltpu.VMEM((ept,), F32),
                    pltpu.VMEM((n1,), F32)]
        extra_in = [tmp_args[0], tmp_args[1]]
    if do_cnt:
        out_type.append(jax.ShapeDtypeStruct((NWORK, n1), F32))
        scratch.append(pltpu.VMEM((n1,), F32))

    @functools.partial(
        pl.kernel,
        mesh=_MESH,
        compiler_params=pltpu.CompilerParams(use_tc_tiling_on_sc=False, needs_layout_passes=False),
        out_type=tuple(out_type),
        scratch_types=scratch,
    )
    def k(*refs):
        pos = [0]

        def take(n):
            r = refs[pos[0]:pos[0] + n]
            pos[0] += n
            return r

        (t1_h, i1_h, t2_h, i2_h) = take(4)
        (a_h, ef_h) = take(2) if do_tmp else (None, None)
        (o1_h, o2_h) = take(2)
        tmp_h = take(1)[0] if do_tmp else None
        cnt_h = take(1)[0] if do_cnt else None
        (idx1, idx2, buf1, buf2, sh1, sh2, gs1, gs2, ss1a, ss1b, ss2a,
         ss2b) = take(12)
        if do_tmp:
            (av, efv, tacc) = take(3)
        if do_cnt:
            cacc = take(1)[0]
        ss1 = (ss1a, ss1b)
        ss2 = (ss2a, ss2b)
        sid = lax.axis_index("s")
        w = _wid()
        base = w * ept
        pltpu.make_async_copy(t1_h.at[pl.ds(sid * n1s, n1s)],
                              sh1.at[pl.ds(sid * n1s, n1s)], gs1).start()
        if stage2:
            pltpu.make_async_copy(t2_h.at[pl.ds(sid * n2s, n2s)],
                                  sh2.at[pl.ds(sid * n2s, n2s)], gs2).start()
        _load_idx_rows(i1_h, idx1, base, nfull, last_off, ss1a)
        _load_idx_rows(i2_h, idx2, base, nfull, last_off, ss2a)

        if do_tmp or do_cnt:
            zero16 = jnp.zeros((16,), F32)
            ones16 = jnp.ones((16,), F32)
            lanes = lax.iota(jnp.int32, 16)
            if do_tmp:
                pltpu.sync_copy(a_h, av)
                pltpu.sync_copy(ef_h.at[pl.ds(base, ept)], efv)

                def zt(i, _):
                    tacc[pl.ds(i * 16, 16)] = zero16
                    return ()

                lax.fori_loop(0, n1 // 16, zt, ())
            if do_cnt:
                def zc(i, _):
                    cacc[pl.ds(i * 16, 16)] = zero16
                    return ()

                lax.fori_loop(0, n1 // 16, zc, ())

            ngrp = (nfull * CH) // 16
            nvec = ngrp + (1 if rem else 0)
            tail_start = ept - 16
            tail_lane0 = tail_start - last_off

            def svec(i, _):
                full = i < ngrp
                j = jnp.where(full, i // (CH // 16), nfull)
                kk = jnp.where(full, (i % (CH // 16)) * 16, tail_lane0)
                estart = jnp.where(full, i * 16, tail_start)
                if rem:
                    msk = jnp.where(full, lanes >= 0, lanes >= (16 - rem))
                else:
                    msk = lanes >= 0
                d = idx1[j, pl.ds(kk, 16)]
                if do_cnt:
                    plsc.addupdate_scatter(cacc, [d], ones16, mask=msk)
                if do_tmp:
                    s = idx2[j, pl.ds(kk, 16)]
                    vals = plsc.load_gather(av, [s]) * efv[pl.ds(estart, 16)]
                    plsc.addupdate_scatter(tacc, [d], vals, mask=msk)
                return ()

            lax.fori_loop(0, nvec, svec, ())

        pltpu.make_async_copy(t1_h.at[pl.ds(0, n1s)],
                              sh1.at[pl.ds(0, n1s)], gs1).wait()
        if stage2:
            pltpu.make_async_copy(t2_h.at[pl.ds(0, n2s)],
                                  sh2.at[pl.ds(0, n2s)], gs2).wait()
        plsc.subcore_barrier()

        def chunk(j, b):
            off = jnp.where(j == nfull, last_off, j * CH)
            g1 = pltpu.make_async_copy(sh1.at[idx1.at[j]], buf1.at[b], gs1)
            t2src = sh2 if stage2 else t2_h
            g2 = pltpu.make_async_copy(t2src.at[idx2.at[j]], buf2.at[b], gs2)
            g1.start()
            g2.start()
            g1.wait()
            pltpu.make_async_copy(
                buf1.at[b], o1_h.at[pl.ds(base + off, CH)], ss1[b]).start()
            g2.wait()
            pltpu.make_async_copy(
                buf2.at[b], o2_h.at[pl.ds(base + off, CH)], ss2[b]).start()

        def wait_stores(b):
            pltpu.make_async_copy(
                buf1.at[b], o1_h.at[pl.ds(base, CH)], ss1[b]).wait()
            pltpu.make_async_copy(
                buf2.at[b], o2_h.at[pl.ds(base, CH)], ss2[b]).wait()

        def loop_body(jj, _):
            for b in range(2):
                j = jj + b

                @pl.when(j < nch)
                def _():
                    @pl.when(j >= 2)
                    def _():
                        wait_stores(b)

                    chunk(j, b)
            return ()

        lax.fori_loop(0, (nch + 1) // 2, lambda q, c: loop_body(2 * q, c),
                      (), unroll=False)
        wait_stores(nch % 2)
        wait_stores((nch + 1) % 2)
        if do_tmp:
            pltpu.sync_copy(tacc, tmp_h.at[w])
        if do_cnt:
            pltpu.sync_copy(cacc, cnt_h.at[w])

    return k(t1, i1, t2, i2, *extra_in)


def sc_segsum(rows, idx, nseg):
    E = idx.shape[0]
    ept = E // NWORK
    nfull = ept // CH
    rem = ept - nfull * CH
    rps = nseg // 16

    @functools.partial(
        pl.kernel,
        mesh=_MESH,
        compiler_params=pltpu.CompilerParams(use_tc_tiling_on_sc=False, needs_layout_passes=False),
        out_type=(jax.ShapeDtypeStruct((nseg, H), F32),
                  jax.ShapeDtypeStruct((nseg, H), F32)),
        scratch_types=[
            pltpu.VMEM((nfull, CH), jnp.int32),
            pltpu.VMEM((8,), jnp.int32),
            pltpu.VMEM((2, CH, H), F32),
            pltpu.VMEM((8, H), F32),
            pltpu.VMEM((rps, H), F32),
            pltpu.VMEM_SHARED((nseg, H), F32),
            pltpu.SemaphoreType.DMA,
            pltpu.SemaphoreType.DMA,
        ],
    )
    def k(rows_h, idx_h, o0_h, o1_h, idx2d, idxt, buf, buft, zbuf, acc,
          lsa, lsb):
        ls = (lsa, lsb)
        cid = lax.axis_index("c")
        sid = lax.axis_index("s")
        base = _wid() * ept

        zero16 = jnp.zeros((16,), F32)

        def zrow(r, _):
            for cc in range(H // 16):
                zbuf[r, pl.ds(cc * 16, 16)] = zero16
            return ()

        lax.fori_loop(0, rps, zrow, ())
        pltpu.sync_copy(zbuf, acc.at[pl.ds(sid * rps, rps)])

        for j in range(nfull):
            pltpu.make_async_copy(idx_h.at[pl.ds(base + j * CH, CH)],
                                  idx2d.at[j], lsa).start()
        if rem:
            pltpu.make_async_copy(idx_h.at[pl.ds(base + nfull * CH, rem)],
                                  idxt, lsa).start()
        for j in range(nfull):
            pltpu.make_async_copy(idx_h.at[pl.ds(base, CH)], idx2d.at[0],
                                  lsa).wait()
        if rem:
            pltpu.make_async_copy(idx_h.at[pl.ds(base + nfull * CH, rem)],
                                  idxt, lsa).wait()

        plsc.subcore_barrier()

        def chunk_load(j, b):
            pltpu.make_async_copy(
                rows_h.at[pl.ds(base + j * CH, CH)], buf.at[b], ls[b]).start()

        def chunk_wait_scatter(j, b):
            pltpu.make_async_copy(
                rows_h.at[pl.ds(base, CH)], buf.at[b], ls[b]).wait()
            pltpu.sync_copy(buf.at[b], acc.at[idx2d.at[j]], add=True)

        chunk_load(0, 0)

        def loop_body(jj, _):
            for b in range(2):
                j = jj + b

                @pl.when(j < nfull)
                def _():
                    @pl.when(j + 1 < nfull)
                    def _():
                        chunk_load(j + 1, (b + 1) % 2)

                    chunk_wait_scatter(j, b)
            return ()

        lax.fori_loop(0, (nfull + 1) // 2, lambda q, c: loop_body(2 * q, c),
                      (), unroll=False)

        if rem:
            pltpu.sync_copy(rows_h.at[pl.ds(base + nfull * CH, rem)], buft)
            pltpu.sync_copy(buft, acc.at[idxt], add=True)

        plsc.subcore_barrier()
        row0 = sid * rps

        @pl.when(cid == 0)
        def _():
            pltpu.sync_copy(acc.at[pl.ds(row0, rps)],
                            o0_h.at[pl.ds(row0, rps)])

        @pl.when(cid == 1)
        def _():
            pltpu.sync_copy(acc.at[pl.ds(row0, rps)],
                            o1_h.at[pl.ds(row0, rps)])

    return k(rows, idx)


def kernel(var_node_features, con_node_features, edge_features_var,
           edge_features_con, rhs, obj, params, edge_index_var,
           edge_index_con, index, num_nodes_var, num_nodes_con):
    p = params
    nv = num_nodes_var.shape[0]
    nc = num_nodes_con.shape[0]
    E = edge_features_var.shape[0]
    e_count = float(E)

    sv = edge_index_var[0]
    dv = edge_index_var[1]
    sc = edge_index_con[0]
    dc = edge_index_con[1]
    efv_flat = edge_features_var.reshape(-1)
    index2d = index.reshape(nc, 1)

    def row(x):
        return x.reshape(1, -1)

    xv0, xc0 = tc_encode(var_node_features, con_node_features,
                         p["venc1_W"], row(p["venc1_b"]), p["venc2_W"],
                         row(p["venc2_b"]), p["cenc1_W"], row(p["cenc1_b"]),
                         p["cenc2_W"], row(p["cenc2_b"]))

    enc_ws = []
    for name in ("varE0", "varE1", "conE0", "conE1"):
        enc_ws.append((row(p[name + "_1_W"][0]), row(p[name + "_1_b"]),
                       p[name + "_2_W"], row(p[name + "_2_b"])))
    estats = tc_edge_stats(edge_features_var, edge_features_con, enc_ws)

    def enc_affine(k, name):
        mu = estats[2 * k] / e_count
        var = estats[2 * k + 1] / e_count - mu * mu
        s = p[name + "_bn_g"] * jax.lax.rsqrt(var + 1e-5)
        t = p[name + "_bn_be"] - mu * s
        return s, t

    xv, xc = xv0, xc0
    xvs = [xv0]
    for i in range(2):
        a, acon, bvar = tc_layer_prep(
            xv, xc, p[f"ass{i}_1_W"], row(p[f"ass{i}_1_b"]),
            p[f"ass{i}_2_W"], row(p[f"ass{i}_2_b"]),
            p[f"varN{i}_1_W"][0:H], p[f"varN{i}_1_W"][H:2 * H],
            row(p[f"varN{i}_1_W"][2 * H]))

        err_args = (rhs, index2d, row(p[f"err{i}_1_W"][0]),
                     row(p[f"err{i}_1_b"]), p[f"err{i}_2_W"],
                     row(p[f"err{i}_2_b"]), row(p[f"err{i}_bn_g"]),
                     row(p[f"err{i}_bn_be"]))

        se, te = enc_affine(i, f"varE{i}")
        w1d = p[f"varN{i}_1_W"][2 * H + 1:]
        wd = se[:, None] * w1d
        c0 = row(te @ w1d + p[f"varN{i}_1_b"])
        if i == 0:
            ag, bg, tmp_parts, cntv = sc_gather_pair(
                acon, dv, bvar, sv, tmp_args=(a.reshape(-1), efv_flat),
                do_cnt=True)
        else:
            ag, bg, tmp_parts = sc_gather_pair(
                acon, dv, bvar, sv, tmp_args=(a.reshape(-1), efv_flat))
        err = tc_err(tmp_parts, *err_args)
        h2, st = tc_edge_main(ag, bg, edge_features_var,
                              enc_ws[i][0], enc_ws[i][1], enc_ws[i][2],
                              enc_ws[i][3], wd, c0, p[f"varN{i}_2_W"],
                              row(p[f"varN{i}_2_b"]))
        s0, s1 = sc_segsum(h2, dv, nc)
        w1x = p[f"conN{i}_1_W"]
        xc, pc, qv = tc_node_varN(s0, s1, cntv, st, row(p[f"varN{i}_bn_g"]),
                                  row(p[f"varN{i}_bn_be"]), err, xv,
                                  w1x[0:H], w1x[H:2 * H], w1x[2 * H:3 * H],
                                  e_count)

        se2, te2 = enc_affine(2 + i, f"conE{i}")
        w1d2 = w1x[3 * H:]
        wd2 = se2[:, None] * w1d2
        c1 = row(te2 @ w1d2 + p[f"conN{i}_1_b"])
        if i == 0:
            qg, pg, cntc = sc_gather_pair(qv, dc, pc, sc, do_cnt=True)
        else:
            qg, pg = sc_gather_pair(qv, dc, pc, sc)
        h2b, st2 = tc_edge_main(qg, pg, edge_features_con,
                                enc_ws[2 + i][0], enc_ws[2 + i][1],
                                enc_ws[2 + i][2], enc_ws[2 + i][3], wd2, c1,
                                p[f"conN{i}_2_W"], row(p[f"conN{i}_2_b"]))
        t0, t1 = sc_segsum(h2b, dc, nv)
        xv = tc_node_conN(t0, t1, cntc, st2, row(p[f"conN{i}_bn_g"]),
                          row(p[f"conN{i}_bn_be"]), e_count)
        xvs.append(xv)

    out = tc_final(xvs[0], xvs[1], xvs[2], p["lin1_W"], row(p["lin1_b"]),
                   p["lin2_W"], row(p["lin2_b"]), p["lin3_W"],
                   row(p["lin3_b"]), p["lin4_W"], row(p["lin4_b"]))
    return out.reshape(-1)


# speedup vs baseline: 1.0373x; 1.0373x over previous
import functools

import jax
import jax.numpy as jnp
from jax import lax
from jax.experimental import pallas as pl
from jax.experimental.pallas import tpu as pltpu
from jax.experimental.pallas import tpu_sc as plsc

H = 64
NB = 32
F32 = jnp.float32
EB = 8000


def _mm(a, b):
    return jnp.dot(a, b, preferred_element_type=F32)


def tc_encode(vnf, cnf, w1v, b1v, w2v, b2v, w1c, b1c, w2c, b2c):
    def body(vnf_r, cnf_r, w1v_r, b1v_r, w2v_r, b2v_r, w1c_r, b1c_r, w2c_r,
             b2c_r, xv0_r, xc0_r):
        xv0_r[...] = _mm(jax.nn.relu(_mm(vnf_r[...], w1v_r[...]) + b1v_r[...]),
                         w2v_r[...]) + b2v_r[...]
        xc0_r[...] = _mm(jax.nn.relu(_mm(cnf_r[...], w1c_r[...]) + b1c_r[...]),
                         w2c_r[...]) + b2c_r[...]

    nv, nc = vnf.shape[0], cnf.shape[0]
    return pl.pallas_call(
        body,
        out_shape=(jax.ShapeDtypeStruct((nv, H), F32),
                   jax.ShapeDtypeStruct((nc, H), F32)),
    )(vnf, cnf, w1v, b1v, w2v, b2v, w1c, b1c, w2c, b2c)


def tc_layer_prep(xv, xc, a1w, a1b, a2w, a2b, w1a, w1b, w1c):
    def body(xv_r, xc_r, a1w_r, a1b_r, a2w_r, a2b_r, w1a_r, w1b_r, w1c_r,
             a_r, acon_r, bvar_r):
        h = jax.nn.relu(_mm(xv_r[...], a1w_r[...]) + a1b_r[...])
        a = jax.nn.sigmoid(_mm(h, a2w_r[...]) + a2b_r[...])
        a_r[...] = a
        acon_r[...] = _mm(xc_r[...], w1a_r[...])
        bvar_r[...] = _mm(xv_r[...], w1b_r[...]) + a * w1c_r[...]

    nv, nc = xv.shape[0], xc.shape[0]
    return pl.pallas_call(
        body,
        out_shape=(jax.ShapeDtypeStruct((nv, 1), F32),
                   jax.ShapeDtypeStruct((nc, H), F32),
                   jax.ShapeDtypeStruct((nv, H), F32)),
    )(xv, xc, a1w, a1b, a2w, a2b, w1a, w1b, w1c)


def tc_edge_stats(efv, efc, ws):
    E = efv.shape[0]
    grid = E // EB

    def body(efv_r, efc_r, *rest):
        wrefs = rest[:16]
        out_r = rest[16]
        i = pl.program_id(0)
        accs = []
        for k in range(4):
            ef = efv_r[...] if k < 2 else efc_r[...]
            w1, b1, w2, b2 = (wrefs[4 * k][...], wrefs[4 * k + 1][...],
                              wrefs[4 * k + 2][...], wrefs[4 * k + 3][...])
            t1 = jax.nn.relu(ef * w1 + b1)
            g = jax.nn.relu(_mm(t1, w2) + b2)
            accs.append(jnp.sum(g, axis=0, keepdims=True))
            accs.append(jnp.sum(g * g, axis=0, keepdims=True))
        upd = jnp.concatenate(accs, axis=0)

        @pl.when(i == 0)
        def _():
            out_r[...] = upd

        @pl.when(i > 0)
        def _():
            out_r[...] = out_r[...] + upd

    flat_ws = [w for enc in ws for w in enc]
    in_specs = [pl.BlockSpec((EB, 1), lambda i: (i, 0)),
                pl.BlockSpec((EB, 1), lambda i: (i, 0))]
    for enc in ws:
        in_specs += [pl.BlockSpec(w.shape, lambda i: (0,) * w.ndim)
                     for w in enc]
    return pl.pallas_call(
        body,
        grid=(grid,),
        in_specs=in_specs,
        out_specs=pl.BlockSpec((8, H), lambda i: (0, 0)),
        out_shape=jax.ShapeDtypeStruct((8, H), F32),
    )(efv, efc, *flat_ws)


def tc_err(tmp_parts, rhs, index2d, w1, b1, w2, b2, bng, bnb):
    nc = rhs.shape[0]

    def body(tp_r, rhs_r, idx_r, w1_r, b1_r, w2_r, b2_r, g_r, be_r, err_r):
        tmp = jnp.sum(tp_r[...], axis=0).reshape(nc, 1)
        u = tmp - rhs_r[...]
        h = jax.nn.relu(u * w1_r[...] + b1_r[...])
        h = jax.nn.relu(_mm(h, w2_r[...]) + b2_r[...])
        mu = jnp.mean(h, axis=0, keepdims=True)
        var = jnp.mean(h * h, axis=0, keepdims=True) - mu * mu
        hbn = (h - mu) * (g_r[...] * jax.lax.rsqrt(var + 1e-5)) + be_r[...]
        m = jnp.max(hbn, axis=0, keepdims=True)
        ex = jnp.exp(hbn - m)
        onehot = (idx_r[...] == jax.lax.broadcasted_iota(
            jnp.int32, (1, NB), 1)).astype(F32)
        gs = jax.lax.dot_general(onehot, ex, (((0,), (0,)), ((), ())),
                                 preferred_element_type=F32)
        denom = _mm(onehot, gs)
        err_r[...] = ex / (denom + 1e-16)

    return pl.pallas_call(
        body,
        out_shape=jax.ShapeDtypeStruct((nc, H), F32),
    )(tmp_parts, rhs, index2d, w1, b1, w2, b2, bng, bnb)


def tc_edge_main(ag, bg, ef, ew1, eb1, ew2, eb2, wd, c0, w2n, b2n):
    E = ag.shape[0]
    grid = E // EB

    def body(ag_r, bg_r, ef_r, ew1_r, eb1_r, ew2_r, eb2_r, wd_r, c0_r, w2n_r,
             b2n_r, h2_r, st_r):
        i = pl.program_id(0)
        t1 = jax.nn.relu(ef_r[...] * ew1_r[...] + eb1_r[...])
        g = jax.nn.relu(_mm(t1, ew2_r[...]) + eb2_r[...])
        h1 = jax.nn.relu(ag_r[...] + bg_r[...] + _mm(g, wd_r[...]) + c0_r[...])
        h2 = jax.nn.relu(_mm(h1, w2n_r[...]) + b2n_r[...])
        h2_r[...] = h2
        upd = jnp.concatenate([jnp.sum(h2, axis=0, keepdims=True),
                               jnp.sum(h2 * h2, axis=0, keepdims=True)],
                              axis=0)

        @pl.when(i == 0)
        def _():
            st_r[...] = upd

        @pl.when(i > 0)
        def _():
            st_r[...] = st_r[...] + upd

    def c(shape):
        return pl.BlockSpec(shape, lambda i: (0,) * len(shape))

    return pl.pallas_call(
        body,
        grid=(grid,),
        in_specs=[pl.BlockSpec((EB, H), lambda i: (i, 0)),
                  pl.BlockSpec((EB, H), lambda i: (i, 0)),
                  pl.BlockSpec((EB, 1), lambda i: (i, 0)),
                  c((1, H)), c((1, H)), c((H, H)), c((1, H)), c((H, H)),
                  c((1, H)), c((H, H)), c((1, H))],
        out_specs=(pl.BlockSpec((EB, H), lambda i: (i, 0)),
                   pl.BlockSpec((2, H), lambda i: (0, 0))),
        out_shape=(jax.ShapeDtypeStruct((E, H), F32),
                   jax.ShapeDtypeStruct((2, H), F32)),
    )(ag, bg, ef, ew1, eb1, ew2, eb2, wd, c0, w2n, b2n)


def tc_node_varN(s0, s1, cnt_parts, stats, bng, bnb, err, xv, w2a, w2b, w2c,
                 e_count):
    nc, nv = s0.shape[0], xv.shape[0]

    def body(s0_r, s1_r, cnt_r, st_r, g_r, be_r, err_r, xv_r, w2a_r, w2b_r,
             w2c_r, xc_r, p_r, q_r):
        cnt = jnp.sum(cnt_r[...], axis=0).reshape(nc, 1)
        mu = st_r[0:1, :] / e_count
        var = st_r[1:2, :] / e_count - mu * mu
        s = g_r[...] * jax.lax.rsqrt(var + 1e-5)
        t = be_r[...] - mu * s
        seg = (s0_r[...] + s1_r[...]) * s + t * cnt
        xc = jax.nn.relu(seg / jnp.maximum(cnt, 1.0))
        xc_r[...] = xc
        p_r[...] = _mm(xc, w2b_r[...]) + _mm(err_r[...], w2c_r[...])
        q_r[...] = _mm(xv_r[...], w2a_r[...])

    return pl.pallas_call(
        body,
        out_shape=(jax.ShapeDtypeStruct((nc, H), F32),
                   jax.ShapeDtypeStruct((nc, H), F32),
                   jax.ShapeDtypeStruct((nv, H), F32)),
    )(s0, s1, cnt_parts, stats, bng, bnb, err, xv, w2a, w2b, w2c)


def tc_node_conN(s0, s1, cnt_parts, stats, bng, bnb, e_count):
    nv = s0.shape[0]

    def body(s0_r, s1_r, cnt_r, st_r, g_r, be_r, xv_r):
        cnt = jnp.sum(cnt_r[...], axis=0).reshape(nv, 1)
        mu = st_r[0:1, :] / e_count
        var = st_r[1:2, :] / e_count - mu * mu
        s = g_r[...] * jax.lax.rsqrt(var + 1e-5)
        t = be_r[...] - mu * s
        seg = (s0_r[...] + s1_r[...]) * s + t * cnt
        xv_r[...] = jax.nn.relu(seg / jnp.maximum(cnt, 1.0))

    return pl.pallas_call(
        body,
        out_shape=jax.ShapeDtypeStruct((nv, H), F32),
    )(s0, s1, cnt_parts, stats, bng, bnb)


def tc_final(xv0, xv1, xv2, w1, b1, w2, b2, w3, b3, w4, b4):
    nv = xv0.shape[0]

    def body(x0_r, x1_r, x2_r, w1_r, b1_r, w2_r, b2_r, w3_r, b3_r, w4_r,
             b4_r, o_r):
        x = jnp.concatenate([x0_r[...], x1_r[...], x2_r[...]], axis=-1)
        x = jax.nn.relu(_mm(x, w1_r[...]) + b1_r[...])
        x = jax.nn.relu(_mm(x, w2_r[...]) + b2_r[...])
        x = jax.nn.relu(_mm(x, w3_r[...]) + b3_r[...])
        o_r[...] = _mm(x, w4_r[...]) + b4_r[...]

    return pl.pallas_call(
        body,
        out_shape=jax.ShapeDtypeStruct((nv, 1), F32),
    )(xv0, xv1, xv2, w1, b1, w2, b2, w3, b3, w4, b4)


_MESH = plsc.VectorSubcoreMesh(core_axis_name="c", subcore_axis_name="s")
NWORK = 32
CH = 128
_SC_PARAMS = pltpu.CompilerParams(use_tc_tiling_on_sc=False,
                                  needs_layout_passes=False)


def _wid():
    return lax.axis_index("s") * 2 + lax.axis_index("c")


def _load_idx_rows(idx_hbm, idx2d, base, nfull, last_off, sem):
    for j in range(nfull):
        pltpu.make_async_copy(idx_hbm.at[pl.ds(base + j * CH, CH)],
                              idx2d.at[j], sem).start()
    pltpu.make_async_copy(idx_hbm.at[pl.ds(base + last_off, CH)],
                          idx2d.at[nfull], sem).start()
    for j in range(nfull + 1):
        pltpu.make_async_copy(idx_hbm.at[pl.ds(base, CH)],
                              idx2d.at[0], sem).wait()


def sc_gather_pair(t1, i1, t2, i2):
    E = i1.shape[0]
    n1, n2 = t1.shape[0], t2.shape[0]
    ept = E // NWORK
    nfull = ept // CH
    rem = ept - nfull * CH
    last_off = ept - CH
    nch = nfull + (1 if rem else 0)

    n1s, n2s = n1 // 16, n2 // 16

    @functools.partial(
        pl.kernel,
        mesh=_MESH,
        compiler_params=_SC_PARAMS,
        out_type=(jax.ShapeDtypeStruct((E, H), F32),
                  jax.ShapeDtypeStruct((E, H), F32)),
        scratch_types=[
            pltpu.VMEM((nch, CH), jnp.int32),
            pltpu.VMEM((nch, CH), jnp.int32),
            pltpu.VMEM((2, CH, H), F32),
            pltpu.VMEM((2, CH, H), F32),
            pltpu.VMEM_SHARED((n1, H), F32),
            pltpu.VMEM_SHARED((n2, H), F32),
            pltpu.SemaphoreType.DMA,
            pltpu.SemaphoreType.DMA,
            pltpu.SemaphoreType.DMA,
            pltpu.SemaphoreType.DMA,
            pltpu.SemaphoreType.DMA,
            pltpu.SemaphoreType.DMA,
        ],
    )
    def k(t1_h, i1_h, t2_h, i2_h, o1_h, o2_h, idx1, idx2, buf1, buf2,
          sh1, sh2, gs1, gs2, ss1a, ss1b, ss2a, ss2b):
        ss1 = (ss1a, ss1b)
        ss2 = (ss2a, ss2b)
        sid = lax.axis_index("s")
        base = _wid() * ept
        pltpu.make_async_copy(t1_h.at[pl.ds(sid * n1s, n1s)],
                              sh1.at[pl.ds(sid * n1s, n1s)], gs1).start()
        pltpu.make_async_copy(t2_h.at[pl.ds(sid * n2s, n2s)],
                              sh2.at[pl.ds(sid * n2s, n2s)], gs2).start()
        _load_idx_rows(i1_h, idx1, base, nfull, last_off, ss1a)
        _load_idx_rows(i2_h, idx2, base, nfull, last_off, ss2a)
        pltpu.make_async_copy(t1_h.at[pl.ds(0, n1s)],
                              sh1.at[pl.ds(0, n1s)], gs1).wait()
        pltpu.make_async_copy(t2_h.at[pl.ds(0, n2s)],
                              sh2.at[pl.ds(0, n2s)], gs2).wait()
        plsc.subcore_barrier()

        def chunk(j, b):
            off = jnp.where(j == nfull, last_off, j * CH)
            g1 = pltpu.make_async_copy(sh1.at[idx1.at[j]], buf1.at[b], gs1)
            g2 = pltpu.make_async_copy(sh2.at[idx2.at[j]], buf2.at[b], gs2)
            g1.start()
            g2.start()
            g1.wait()
            pltpu.make_async_copy(
                buf1.at[b], o1_h.at[pl.ds(base + off, CH)], ss1[b]).start()
            g2.wait()
            pltpu.make_async_copy(
                buf2.at[b], o2_h.at[pl.ds(base + off, CH)], ss2[b]).start()

        def wait_stores(b):
            pltpu.make_async_copy(
                buf1.at[b], o1_h.at[pl.ds(base, CH)], ss1[b]).wait()
            pltpu.make_async_copy(
                buf2.at[b], o2_h.at[pl.ds(base, CH)], ss2[b]).wait()

        def loop_body(jj, _):
            for b in range(2):
                j = jj + b

                @pl.when(j < nch)
                def _():
                    @pl.when(j >= 2)
                    def _():
                        wait_stores(b)

                    chunk(j, b)
            return ()

        lax.fori_loop(0, (nch + 1) // 2, lambda q, c: loop_body(2 * q, c),
                      (), unroll=False)
        wait_stores(nch % 2)
        wait_stores((nch + 1) % 2)

    return k(t1, i1, t2, i2)


def sc_segsum(rows, idx, nseg):
    E = idx.shape[0]
    ept = E // NWORK
    nfull = ept // CH
    rem = ept - nfull * CH
    rps = nseg // 16

    @functools.partial(
        pl.kernel,
        mesh=_MESH,
        compiler_params=_SC_PARAMS,
        out_type=(jax.ShapeDtypeStruct((nseg, H), F32),
                  jax.ShapeDtypeStruct((nseg, H), F32)),
        scratch_types=[
            pltpu.VMEM((nfull, CH), jnp.int32),
            pltpu.VMEM((8,), jnp.int32),
            pltpu.VMEM((2, CH, H), F32),
            pltpu.VMEM((8, H), F32),
            pltpu.VMEM((rps, H), F32),
            pltpu.VMEM_SHARED((nseg, H), F32),
            pltpu.SemaphoreType.DMA,
            pltpu.SemaphoreType.DMA,
        ],
    )
    def k(rows_h, idx_h, o0_h, o1_h, idx2d, idxt, buf, buft, zbuf, acc,
          lsa, lsb):
        ls = (lsa, lsb)
        cid = lax.axis_index("c")
        sid = lax.axis_index("s")
        base = _wid() * ept

        zero16 = jnp.zeros((16,), F32)

        def zrow(r, _):
            for cc in range(H // 16):
                zbuf[r, pl.ds(cc * 16, 16)] = zero16
            return ()

        lax.fori_loop(0, rps, zrow, ())
        pltpu.sync_copy(zbuf, acc.at[pl.ds(sid * rps, rps)])

        for j in range(nfull):
            pltpu.make_async_copy(idx_h.at[pl.ds(base + j * CH, CH)],
                                  idx2d.at[j], lsa).start()
        if rem:
            pltpu.make_async_copy(idx_h.at[pl.ds(base + nfull * CH, rem)],
                                  idxt, lsa).start()
        for j in range(nfull):
            pltpu.make_async_copy(idx_h.at[pl.ds(base, CH)], idx2d.at[0],
                                  lsa).wait()
        if rem:
            pltpu.make_async_copy(idx_h.at[pl.ds(base + nfull * CH, rem)],
                                  idxt, lsa).wait()

        plsc.subcore_barrier()

        def chunk_load(j, b):
            pltpu.make_async_copy(
                rows_h.at[pl.ds(base + j * CH, CH)], buf.at[b], ls[b]).start()

        def chunk_wait_scatter(j, b):
            pltpu.make_async_copy(
                rows_h.at[pl.ds(base, CH)], buf.at[b], ls[b]).wait()
            pltpu.sync_copy(buf.at[b], acc.at[idx2d.at[j]], add=True)

        chunk_load(0, 0)

        def loop_body(jj, _):
            for b in range(2):
                j = jj + b

                @pl.when(j < nfull)
                def _():
                    @pl.when(j + 1 < nfull)
                    def _():
                        chunk_load(j + 1, (b + 1) % 2)

                    chunk_wait_scatter(j, b)
            return ()

        lax.fori_loop(0, (nfull + 1) // 2, lambda q, c: loop_body(2 * q, c),
                      (), unroll=False)

        if rem:
            pltpu.sync_copy(rows_h.at[pl.ds(base + nfull * CH, rem)], buft)
            pltpu.sync_copy(buft, acc.at[idxt], add=True)

        plsc.subcore_barrier()
        row0 = sid * rps

        @pl.when(cid == 0)
        def _():
            pltpu.sync_copy(acc.at[pl.ds(row0, rps)],
                            o0_h.at[pl.ds(row0, rps)])

        @pl.when(cid == 1)
        def _():
            pltpu.sync_copy(acc.at[pl.ds(row0, rps)],
                            o1_h.at[pl.ds(row0, rps)])

    return k(rows, idx)


def sc_counts(dv, dc, nc, nv):
    E = dv.shape[0]
    ept = E // NWORK
    niter = (ept + 15) // 16

    @functools.partial(
        pl.kernel,
        mesh=_MESH,
        compiler_params=_SC_PARAMS,
        out_type=(jax.ShapeDtypeStruct((NWORK, nc), F32),
                  jax.ShapeDtypeStruct((NWORK, nv), F32)),
        scratch_types=[
            pltpu.VMEM((ept,), jnp.int32),
            pltpu.VMEM((ept,), jnp.int32),
            pltpu.VMEM((nc,), F32),
            pltpu.VMEM((nv,), F32),
        ],
    )
    def k(dv_h, dc_h, ov_h, oc_h, dvv, dcv, accv, accc):
        w = _wid()
        base = w * ept
        pltpu.sync_copy(dv_h.at[pl.ds(base, ept)], dvv)
        pltpu.sync_copy(dc_h.at[pl.ds(base, ept)], dcv)
        zero16 = jnp.zeros((16,), F32)
        for n, ref in ((nc, accv), (nv, accc)):
            def zbody(i, _, ref=ref):
                ref[pl.ds(i * 16, 16)] = zero16
                return ()
            lax.fori_loop(0, n // 16, zbody, ())
        ones = jnp.ones((16,), F32)
        lanes = lax.iota(jnp.int32, 16)

        def body(i, _):
            msk = lanes < (ept - i * 16)
            iv = dvv[pl.ds(i * 16, 16)]
            plsc.addupdate_scatter(accv, [iv], ones, mask=msk)
            ic = dcv[pl.ds(i * 16, 16)]
            plsc.addupdate_scatter(accc, [ic], ones, mask=msk)
            return ()

        lax.fori_loop(0, niter, body, ())
        pltpu.sync_copy(accv, ov_h.at[w])
        pltpu.sync_copy(accc, oc_h.at[w])

    return k(dv, dc)


def sc_tmp(a_vec, sv, dv, efv, nc, nv):
    E = sv.shape[0]
    ept = E // NWORK
    niter = (ept + 15) // 16

    @functools.partial(
        pl.kernel,
        mesh=_MESH,
        compiler_params=_SC_PARAMS,
        out_type=jax.ShapeDtypeStruct((NWORK, nc), F32),
        scratch_types=[
            pltpu.VMEM((nv,), F32),
            pltpu.VMEM((ept,), jnp.int32),
            pltpu.VMEM((ept,), jnp.int32),
            pltpu.VMEM((ept,), F32),
            pltpu.VMEM((nc,), F32),
        ],
    )
    def k(a_h, sv_h, dv_h, ef_h, o_h, av, svv, dvv, efvv, acc):
        w = _wid()
        base = w * ept
        pltpu.sync_copy(a_h, av)
        pltpu.sync_copy(sv_h.at[pl.ds(base, ept)], svv)
        pltpu.sync_copy(dv_h.at[pl.ds(base, ept)], dvv)
        pltpu.sync_copy(ef_h.at[pl.ds(base, ept)], efvv)
        zero16 = jnp.zeros((16,), F32)

        def zbody(i, _):
            acc[pl.ds(i * 16, 16)] = zero16
            return ()

        lax.fori_loop(0, nc // 16, zbody, ())
        lanes = lax.iota(jnp.int32, 16)

        def body(i, _):
            msk = lanes < (ept - i * 16)
            si = svv[pl.ds(i * 16, 16)]
            vals = plsc.load_gather(av, [si]) * efvv[pl.ds(i * 16, 16)]
            di = dvv[pl.ds(i * 16, 16)]
            plsc.addupdate_scatter(acc, [di], vals, mask=msk)
            return ()

        lax.fori_loop(0, niter, body, ())
        pltpu.sync_copy(acc, o_h.at[w])

    return k(a_vec, sv, dv, efv)


def kernel(var_node_features, con_node_features, edge_features_var,
           edge_features_con, rhs, obj, params, edge_index_var,
           edge_index_con, index, num_nodes_var, num_nodes_con):
    p = params
    nv = num_nodes_var.shape[0]
    nc = num_nodes_con.shape[0]
    E = edge_features_var.shape[0]
    e_count = float(E)

    sv = edge_index_var[0]
    dv = edge_index_var[1]
    sc = edge_index_con[0]
    dc = edge_index_con[1]
    efv_flat = edge_features_var.reshape(-1)
    index2d = index.reshape(nc, 1)

    def row(x):
        return x.reshape(1, -1)

    xv0, xc0 = tc_encode(var_node_features, con_node_features,
                         p["venc1_W"], row(p["venc1_b"]), p["venc2_W"],
                         row(p["venc2_b"]), p["cenc1_W"], row(p["cenc1_b"]),
                         p["cenc2_W"], row(p["cenc2_b"]))

    cntv_parts, cntc_parts = sc_counts(dv, dc, nc, nv)

    enc_ws = []
    for name in ("varE0", "varE1", "conE0", "conE1"):
        enc_ws.append((row(p[name + "_1_W"][0]), row(p[name + "_1_b"]),
                       p[name + "_2_W"], row(p[name + "_2_b"])))
    estats = tc_edge_stats(edge_features_var, edge_features_con, enc_ws)

    def enc_affine(k, name):
        mu = estats[2 * k] / e_count
        var = estats[2 * k + 1] / e_count - mu * mu
        s = p[name + "_bn_g"] * jax.lax.rsqrt(var + 1e-5)
        t = p[name + "_bn_be"] - mu * s
        return s, t

    xv, xc = xv0, xc0
    xvs = [xv0]
    for i in range(2):
        a, acon, bvar = tc_layer_prep(
            xv, xc, p[f"ass{i}_1_W"], row(p[f"ass{i}_1_b"]),
            p[f"ass{i}_2_W"], row(p[f"ass{i}_2_b"]),
            p[f"varN{i}_1_W"][0:H], p[f"varN{i}_1_W"][H:2 * H],
            row(p[f"varN{i}_1_W"][2 * H]))

        tmp_parts = sc_tmp(a.reshape(-1), sv, dv, efv_flat, nc, nv)
        err = tc_err(tmp_parts, rhs, index2d, row(p[f"err{i}_1_W"][0]),
                     row(p[f"err{i}_1_b"]), p[f"err{i}_2_W"],
                     row(p[f"err{i}_2_b"]), row(p[f"err{i}_bn_g"]),
                     row(p[f"err{i}_bn_be"]))

        se, te = enc_affine(i, f"varE{i}")
        w1d = p[f"varN{i}_1_W"][2 * H + 1:]
        wd = se[:, None] * w1d
        c0 = row(te @ w1d + p[f"varN{i}_1_b"])
        ag, bg = sc_gather_pair(acon, dv, bvar, sv)
        h2, st = tc_edge_main(ag, bg, edge_features_var,
                              enc_ws[i][0], enc_ws[i][1], enc_ws[i][2],
                              enc_ws[i][3], wd, c0, p[f"varN{i}_2_W"],
                              row(p[f"varN{i}_2_b"]))
        s0, s1 = sc_segsum(h2, dv, nc)
        w1x = p[f"conN{i}_1_W"]
        xc, pc, qv = tc_node_varN(s0, s1, cntv_parts, st,
                                  row(p[f"varN{i}_bn_g"]),
                                  row(p[f"varN{i}_bn_be"]), err, xv,
                                  w1x[0:H], w1x[H:2 * H], w1x[2 * H:3 * H],
                                  e_count)

        se2, te2 = enc_affine(2 + i, f"conE{i}")
        w1d2 = w1x[3 * H:]
        wd2 = se2[:, None] * w1d2
        c1 = row(te2 @ w1d2 + p[f"conN{i}_1_b"])
        qg, pg = sc_gather_pair(qv, dc, pc, sc)
        h2b, st2 = tc_edge_main(qg, pg, edge_features_con,
                                enc_ws[2 + i][0], enc_ws[2 + i][1],
                                enc_ws[2 + i][2], enc_ws[2 + i][3], wd2, c1,
                                p[f"conN{i}_2_W"], row(p[f"conN{i}_2_b"]))
        t0, t1 = sc_segsum(h2b, dc, nv)
        xv = tc_node_conN(t0, t1, cntc_parts, st2, row(p[f"conN{i}_bn_g"]),
                          row(p[f"conN{i}_bn_be"]), e_count)
        xvs.append(xv)

    out = tc_final(xvs[0], xvs[1], xvs[2], p["lin1_W"], row(p["lin1_b"]),
                   p["lin2_W"], row(p["lin2_b"]), p["lin3_W"],
                   row(p["lin3_b"]), p["lin4_W"], row(p["lin4_b"]))
    return out.reshape(-1)


# speedup vs baseline: 1.0385x; 1.0012x over previous
import functools

import jax
import jax.numpy as jnp
from jax import lax
from jax.experimental import pallas as pl
from jax.experimental.pallas import tpu as pltpu
from jax.experimental.pallas import tpu_sc as plsc

H = 64
NB = 32
F32 = jnp.float32
EB = 10000


def _mm(a, b):
    return jnp.dot(a, b, preferred_element_type=F32)


def tc_encode(vnf, cnf, w1v, b1v, w2v, b2v, w1c, b1c, w2c, b2c):
    def body(vnf_r, cnf_r, w1v_r, b1v_r, w2v_r, b2v_r, w1c_r, b1c_r, w2c_r,
             b2c_r, xv0_r, xc0_r):
        xv0_r[...] = _mm(jax.nn.relu(_mm(vnf_r[...], w1v_r[...]) + b1v_r[...]),
                         w2v_r[...]) + b2v_r[...]
        xc0_r[...] = _mm(jax.nn.relu(_mm(cnf_r[...], w1c_r[...]) + b1c_r[...]),
                         w2c_r[...]) + b2c_r[...]

    nv, nc = vnf.shape[0], cnf.shape[0]
    return pl.pallas_call(
        body,
        out_shape=(jax.ShapeDtypeStruct((nv, H), F32),
                   jax.ShapeDtypeStruct((nc, H), F32)),
    )(vnf, cnf, w1v, b1v, w2v, b2v, w1c, b1c, w2c, b2c)


def tc_layer_prep(xv, xc, a1w, a1b, a2w, a2b, w1a, w1b, w1c):
    def body(xv_r, xc_r, a1w_r, a1b_r, a2w_r, a2b_r, w1a_r, w1b_r, w1c_r,
             a_r, acon_r, bvar_r):
        h = jax.nn.relu(_mm(xv_r[...], a1w_r[...]) + a1b_r[...])
        a = jax.nn.sigmoid(_mm(h, a2w_r[...]) + a2b_r[...])
        a_r[...] = a
        acon_r[...] = _mm(xc_r[...], w1a_r[...])
        bvar_r[...] = _mm(xv_r[...], w1b_r[...]) + a * w1c_r[...]

    nv, nc = xv.shape[0], xc.shape[0]
    return pl.pallas_call(
        body,
        out_shape=(jax.ShapeDtypeStruct((nv, 1), F32),
                   jax.ShapeDtypeStruct((nc, H), F32),
                   jax.ShapeDtypeStruct((nv, H), F32)),
    )(xv, xc, a1w, a1b, a2w, a2b, w1a, w1b, w1c)


def tc_edge_stats(efv, efc, ws):
    E = efv.shape[0]
    grid = E // EB

    def body(efv_r, efc_r, *rest):
        wrefs = rest[:16]
        out_r = rest[16]
        i = pl.program_id(0)
        accs = []
        for k in range(4):
            ef = efv_r[...] if k < 2 else efc_r[...]
            w1, b1, w2, b2 = (wrefs[4 * k][...], wrefs[4 * k + 1][...],
                              wrefs[4 * k + 2][...], wrefs[4 * k + 3][...])
            t1 = jax.nn.relu(ef * w1 + b1)
            g = jax.nn.relu(_mm(t1, w2) + b2)
            accs.append(jnp.sum(g, axis=0, keepdims=True))
            accs.append(jnp.sum(g * g, axis=0, keepdims=True))
        upd = jnp.concatenate(accs, axis=0)

        @pl.when(i == 0)
        def _():
            out_r[...] = upd

        @pl.when(i > 0)
        def _():
            out_r[...] = out_r[...] + upd

    flat_ws = [w for enc in ws for w in enc]
    in_specs = [pl.BlockSpec((EB, 1), lambda i: (i, 0)),
                pl.BlockSpec((EB, 1), lambda i: (i, 0))]
    for enc in ws:
        in_specs += [pl.BlockSpec(w.shape, lambda i: (0,) * w.ndim)
                     for w in enc]
    return pl.pallas_call(
        body,
        grid=(grid,),
        in_specs=in_specs,
        out_specs=pl.BlockSpec((8, H), lambda i: (0, 0)),
        out_shape=jax.ShapeDtypeStruct((8, H), F32),
    )(efv, efc, *flat_ws)


def tc_err(tmp_parts, rhs, index2d, w1, b1, w2, b2, bng, bnb):
    nc = rhs.shape[0]

    def body(tp_r, rhs_r, idx_r, w1_r, b1_r, w2_r, b2_r, g_r, be_r, err_r):
        tmp = jnp.sum(tp_r[...], axis=0).reshape(nc, 1)
        u = tmp - rhs_r[...]
        h = jax.nn.relu(u * w1_r[...] + b1_r[...])
        h = jax.nn.relu(_mm(h, w2_r[...]) + b2_r[...])
        mu = jnp.mean(h, axis=0, keepdims=True)
        var = jnp.mean(h * h, axis=0, keepdims=True) - mu * mu
        hbn = (h - mu) * (g_r[...] * jax.lax.rsqrt(var + 1e-5)) + be_r[...]
        m = jnp.max(hbn, axis=0, keepdims=True)
        ex = jnp.exp(hbn - m)
        onehot = (idx_r[...] == jax.lax.broadcasted_iota(
            jnp.int32, (1, NB), 1)).astype(F32)
        gs = jax.lax.dot_general(onehot, ex, (((0,), (0,)), ((), ())),
                                 preferred_element_type=F32)
        denom = _mm(onehot, gs)
        err_r[...] = ex / (denom + 1e-16)

    return pl.pallas_call(
        body,
        out_shape=jax.ShapeDtypeStruct((nc, H), F32),
    )(tmp_parts, rhs, index2d, w1, b1, w2, b2, bng, bnb)


def tc_edge_main(ag, bg, ef, ew1, eb1, ew2, eb2, wd, c0, w2n, b2n):
    E = ag.shape[0]
    grid = E // EB

    def body(ag_r, bg_r, ef_r, ew1_r, eb1_r, ew2_r, eb2_r, wd_r, c0_r, w2n_r,
             b2n_r, h2_r, st_r):
        i = pl.program_id(0)
        t1 = jax.nn.relu(ef_r[...] * ew1_r[...] + eb1_r[...])
        g = jax.nn.relu(_mm(t1, ew2_r[...]) + eb2_r[...])
        h1 = jax.nn.relu(ag_r[...] + bg_r[...] + _mm(g, wd_r[...]) + c0_r[...])
        h2 = jax.nn.relu(_mm(h1, w2n_r[...]) + b2n_r[...])
        h2_r[...] = h2
        upd = jnp.concatenate([jnp.sum(h2, axis=0, keepdims=True),
                               jnp.sum(h2 * h2, axis=0, keepdims=True)],
                              axis=0)

        @pl.when(i == 0)
        def _():
            st_r[...] = upd

        @pl.when(i > 0)
        def _():
            st_r[...] = st_r[...] + upd

    def c(shape):
        return pl.BlockSpec(shape, lambda i: (0,) * len(shape))

    return pl.pallas_call(
        body,
        grid=(grid,),
        in_specs=[pl.BlockSpec((EB, H), lambda i: (i, 0)),
                  pl.BlockSpec((EB, H), lambda i: (i, 0)),
                  pl.BlockSpec((EB, 1), lambda i: (i, 0)),
                  c((1, H)), c((1, H)), c((H, H)), c((1, H)), c((H, H)),
                  c((1, H)), c((H, H)), c((1, H))],
        out_specs=(pl.BlockSpec((EB, H), lambda i: (i, 0)),
                   pl.BlockSpec((2, H), lambda i: (0, 0))),
        out_shape=(jax.ShapeDtypeStruct((E, H), F32),
                   jax.ShapeDtypeStruct((2, H), F32)),
    )(ag, bg, ef, ew1, eb1, ew2, eb2, wd, c0, w2n, b2n)


def tc_node_varN(s0, s1, cnt_parts, stats, bng, bnb, err, xv, w2a, w2b, w2c,
                 e_count):
    nc, nv = s0.shape[0], xv.shape[0]

    def body(s0_r, s1_r, cnt_r, st_r, g_r, be_r, err_r, xv_r, w2a_r, w2b_r,
             w2c_r, xc_r, p_r, q_r):
        cnt = jnp.sum(cnt_r[...], axis=0).reshape(nc, 1)
        mu = st_r[0:1, :] / e_count
        var = st_r[1:2, :] / e_count - mu * mu
        s = g_r[...] * jax.lax.rsqrt(var + 1e-5)
        t = be_r[...] - mu * s
        seg = (s0_r[...] + s1_r[...]) * s + t * cnt
        xc = jax.nn.relu(seg / jnp.maximum(cnt, 1.0))
        xc_r[...] = xc
        p_r[...] = _mm(xc, w2b_r[...]) + _mm(err_r[...], w2c_r[...])
        q_r[...] = _mm(xv_r[...], w2a_r[...])

    return pl.pallas_call(
        body,
        out_shape=(jax.ShapeDtypeStruct((nc, H), F32),
                   jax.ShapeDtypeStruct((nc, H), F32),
                   jax.ShapeDtypeStruct((nv, H), F32)),
    )(s0, s1, cnt_parts, stats, bng, bnb, err, xv, w2a, w2b, w2c)


def tc_node_conN(s0, s1, cnt_parts, stats, bng, bnb, e_count):
    nv = s0.shape[0]

    def body(s0_r, s1_r, cnt_r, st_r, g_r, be_r, xv_r):
        cnt = jnp.sum(cnt_r[...], axis=0).reshape(nv, 1)
        mu = st_r[0:1, :] / e_count
        var = st_r[1:2, :] / e_count - mu * mu
        s = g_r[...] * jax.lax.rsqrt(var + 1e-5)
        t = be_r[...] - mu * s
        seg = (s0_r[...] + s1_r[...]) * s + t * cnt
        xv_r[...] = jax.nn.relu(seg / jnp.maximum(cnt, 1.0))

    return pl.pallas_call(
        body,
        out_shape=jax.ShapeDtypeStruct((nv, H), F32),
    )(s0, s1, cnt_parts, stats, bng, bnb)


def tc_final(xv0, xv1, xv2, w1, b1, w2, b2, w3, b3, w4, b4):
    nv = xv0.shape[0]

    def body(x0_r, x1_r, x2_r, w1_r, b1_r, w2_r, b2_r, w3_r, b3_r, w4_r,
             b4_r, o_r):
        x = jnp.concatenate([x0_r[...], x1_r[...], x2_r[...]], axis=-1)
        x = jax.nn.relu(_mm(x, w1_r[...]) + b1_r[...])
        x = jax.nn.relu(_mm(x, w2_r[...]) + b2_r[...])
        x = jax.nn.relu(_mm(x, w3_r[...]) + b3_r[...])
        o_r[...] = _mm(x, w4_r[...]) + b4_r[...]

    return pl.pallas_call(
        body,
        out_shape=jax.ShapeDtypeStruct((nv, 1), F32),
    )(xv0, xv1, xv2, w1, b1, w2, b2, w3, b3, w4, b4)


_MESH = plsc.VectorSubcoreMesh(core_axis_name="c", subcore_axis_name="s")
NWORK = 32
CH = 128
_SC_PARAMS = pltpu.CompilerParams(use_tc_tiling_on_sc=False,
                                  needs_layout_passes=False)


def _wid():
    return lax.axis_index("s") * 2 + lax.axis_index("c")


def _load_idx_rows(idx_hbm, idx2d, base, nfull, last_off, sem):
    for j in range(nfull):
        pltpu.make_async_copy(idx_hbm.at[pl.ds(base + j * CH, CH)],
                              idx2d.at[j], sem).start()
    pltpu.make_async_copy(idx_hbm.at[pl.ds(base + last_off, CH)],
                          idx2d.at[nfull], sem).start()
    for j in range(nfull + 1):
        pltpu.make_async_copy(idx_hbm.at[pl.ds(base, CH)],
                              idx2d.at[0], sem).wait()


def sc_gather_pair(t1, i1, t2, i2):
    E = i1.shape[0]
    n1, n2 = t1.shape[0], t2.shape[0]
    ept = E // NWORK
    nfull = ept // CH
    rem = ept - nfull * CH
    last_off = ept - CH
    nch = nfull + (1 if rem else 0)

    n1s, n2s = n1 // 16, n2 // 16

    @functools.partial(
        pl.kernel,
        mesh=_MESH,
        compiler_params=_SC_PARAMS,
        out_type=(jax.ShapeDtypeStruct((E, H), F32),
                  jax.ShapeDtypeStruct((E, H), F32)),
        scratch_types=[
            pltpu.VMEM((nch, CH), jnp.int32),
            pltpu.VMEM((nch, CH), jnp.int32),
            pltpu.VMEM((2, CH, H), F32),
            pltpu.VMEM((2, CH, H), F32),
            pltpu.VMEM_SHARED((n1, H), F32),
            pltpu.VMEM_SHARED((n2, H), F32),
            pltpu.SemaphoreType.DMA,
            pltpu.SemaphoreType.DMA,
            pltpu.SemaphoreType.DMA,
            pltpu.SemaphoreType.DMA,
            pltpu.SemaphoreType.DMA,
            pltpu.SemaphoreType.DMA,
        ],
    )
    def k(t1_h, i1_h, t2_h, i2_h, o1_h, o2_h, idx1, idx2, buf1, buf2,
          sh1, sh2, gs1, gs2, ss1a, ss1b, ss2a, ss2b):
        ss1 = (ss1a, ss1b)
        ss2 = (ss2a, ss2b)
        sid = lax.axis_index("s")
        base = _wid() * ept
        pltpu.make_async_copy(t1_h.at[pl.ds(sid * n1s, n1s)],
                              sh1.at[pl.ds(sid * n1s, n1s)], gs1).start()
        pltpu.make_async_copy(t2_h.at[pl.ds(sid * n2s, n2s)],
                              sh2.at[pl.ds(sid * n2s, n2s)], gs2).start()
        _load_idx_rows(i1_h, idx1, base, nfull, last_off, ss1a)
        _load_idx_rows(i2_h, idx2, base, nfull, last_off, ss2a)
        pltpu.make_async_copy(t1_h.at[pl.ds(0, n1s)],
                              sh1.at[pl.ds(0, n1s)], gs1).wait()
        pltpu.make_async_copy(t2_h.at[pl.ds(0, n2s)],
                              sh2.at[pl.ds(0, n2s)], gs2).wait()
        plsc.subcore_barrier()

        def chunk(j, b):
            off = jnp.where(j == nfull, last_off, j * CH)
            g1 = pltpu.make_async_copy(sh1.at[idx1.at[j]], buf1.at[b], gs1)
            g2 = pltpu.make_async_copy(sh2.at[idx2.at[j]], buf2.at[b], gs2)
            g1.start()
            g2.start()
            g1.wait()
            pltpu.make_async_copy(
                buf1.at[b], o1_h.at[pl.ds(base + off, CH)], ss1[b]).start()
            g2.wait()
            pltpu.make_async_copy(
                buf2.at[b], o2_h.at[pl.ds(base + off, CH)], ss2[b]).start()

        def wait_stores(b):
            pltpu.make_async_copy(
                buf1.at[b], o1_h.at[pl.ds(base, CH)], ss1[b]).wait()
            pltpu.make_async_copy(
                buf2.at[b], o2_h.at[pl.ds(base, CH)], ss2[b]).wait()

        def loop_body(jj, _):
            for b in range(2):
                j = jj + b

                @pl.when(j < nch)
                def _():
                    @pl.when(j >= 2)
                    def _():
                        wait_stores(b)

                    chunk(j, b)
            return ()

        lax.fori_loop(0, (nch + 1) // 2, lambda q, c: loop_body(2 * q, c),
                      (), unroll=False)
        wait_stores(nch % 2)
        wait_stores((nch + 1) % 2)

    return k(t1, i1, t2, i2)


def sc_segsum(rows, idx, nseg):
    E = idx.shape[0]
    ept = E // NWORK
    nfull = ept // CH
    rem = ept - nfull * CH
    rps = nseg // 16

    @functools.partial(
        pl.kernel,
        mesh=_MESH,
        compiler_params=_SC_PARAMS,
        out_type=(jax.ShapeDtypeStruct((nseg, H), F32),
                  jax.ShapeDtypeStruct((nseg, H), F32)),
        scratch_types=[
            pltpu.VMEM((nfull, CH), jnp.int32),
            pltpu.VMEM((8,), jnp.int32),
            pltpu.VMEM((2, CH, H), F32),
            pltpu.VMEM((8, H), F32),
            pltpu.VMEM((rps, H), F32),
            pltpu.VMEM_SHARED((nseg, H), F32),
            pltpu.SemaphoreType.DMA,
            pltpu.SemaphoreType.DMA,
        ],
    )
    def k(rows_h, idx_h, o0_h, o1_h, idx2d, idxt, buf, buft, zbuf, acc,
          lsa, lsb):
        ls = (lsa, lsb)
        cid = lax.axis_index("c")
        sid = lax.axis_index("s")
        base = _wid() * ept

        zero16 = jnp.zeros((16,), F32)

        def zrow(r, _):
            for cc in range(H // 16):
                zbuf[r, pl.ds(cc * 16, 16)] = zero16
            return ()

        lax.fori_loop(0, rps, zrow, ())
        pltpu.sync_copy(zbuf, acc.at[pl.ds(sid * rps, rps)])

        for j in range(nfull):
            pltpu.make_async_copy(idx_h.at[pl.ds(base + j * CH, CH)],
                                  idx2d.at[j], lsa).start()
        if rem:
            pltpu.make_async_copy(idx_h.at[pl.ds(base + nfull * CH, rem)],
                                  idxt, lsa).start()
        for j in range(nfull):
            pltpu.make_async_copy(idx_h.at[pl.ds(base, CH)], idx2d.at[0],
                                  lsa).wait()
        if rem:
            pltpu.make_async_copy(idx_h.at[pl.ds(base + nfull * CH, rem)],
                                  idxt, lsa).wait()

        plsc.subcore_barrier()

        def chunk_load(j, b):
            pltpu.make_async_copy(
                rows_h.at[pl.ds(base + j * CH, CH)], buf.at[b], ls[b]).start()

        def chunk_wait_scatter(j, b):
            pltpu.make_async_copy(
                rows_h.at[pl.ds(base, CH)], buf.at[b], ls[b]).wait()
            pltpu.sync_copy(buf.at[b], acc.at[idx2d.at[j]], add=True)

        chunk_load(0, 0)

        def loop_body(jj, _):
            for b in range(2):
                j = jj + b

                @pl.when(j < nfull)
                def _():
                    @pl.when(j + 1 < nfull)
                    def _():
                        chunk_load(j + 1, (b + 1) % 2)

                    chunk_wait_scatter(j, b)
            return ()

        lax.fori_loop(0, (nfull + 1) // 2, lambda q, c: loop_body(2 * q, c),
                      (), unroll=False)

        if rem:
            pltpu.sync_copy(rows_h.at[pl.ds(base + nfull * CH, rem)], buft)
            pltpu.sync_copy(buft, acc.at[idxt], add=True)

        plsc.subcore_barrier()
        row0 = sid * rps

        @pl.when(cid == 0)
        def _():
            pltpu.sync_copy(acc.at[pl.ds(row0, rps)],
                            o0_h.at[pl.ds(row0, rps)])

        @pl.when(cid == 1)
        def _():
            pltpu.sync_copy(acc.at[pl.ds(row0, rps)],
                            o1_h.at[pl.ds(row0, rps)])

    return k(rows, idx)


def sc_counts(dv, dc, nc, nv):
    E = dv.shape[0]
    ept = E // NWORK
    niter = (ept + 15) // 16

    @functools.partial(
        pl.kernel,
        mesh=_MESH,
        compiler_params=_SC_PARAMS,
        out_type=(jax.ShapeDtypeStruct((NWORK, nc), F32),
                  jax.ShapeDtypeStruct((NWORK, nv), F32)),
        scratch_types=[
            pltpu.VMEM((ept,), jnp.int32),
            pltpu.VMEM((ept,), jnp.int32),
            pltpu.VMEM((nc,), F32),
            pltpu.VMEM((nv,), F32),
        ],
    )
    def k(dv_h, dc_h, ov_h, oc_h, dvv, dcv, accv, accc):
        w = _wid()
        base = w * ept
        pltpu.sync_copy(dv_h.at[pl.ds(base, ept)], dvv)
        pltpu.sync_copy(dc_h.at[pl.ds(base, ept)], dcv)
        zero16 = jnp.zeros((16,), F32)
        for n, ref in ((nc, accv), (nv, accc)):
            def zbody(i, _, ref=ref):
                ref[pl.ds(i * 16, 16)] = zero16
                return ()
            lax.fori_loop(0, n // 16, zbody, ())
        ones = jnp.ones((16,), F32)
        lanes = lax.iota(jnp.int32, 16)

        def body(i, _):
            msk = lanes < (ept - i * 16)
            iv = dvv[pl.ds(i * 16, 16)]
            plsc.addupdate_scatter(accv, [iv], ones, mask=msk)
            ic = dcv[pl.ds(i * 16, 16)]
            plsc.addupdate_scatter(accc, [ic], ones, mask=msk)
            return ()

        lax.fori_loop(0, niter, body, ())
        pltpu.sync_copy(accv, ov_h.at[w])
        pltpu.sync_copy(accc, oc_h.at[w])

    return k(dv, dc)


def sc_tmp(a_vec, sv, dv, efv, nc, nv):
    E = sv.shape[0]
    ept = E // NWORK
    niter = (ept + 15) // 16

    @functools.partial(
        pl.kernel,
        mesh=_MESH,
        compiler_params=_SC_PARAMS,
        out_type=jax.ShapeDtypeStruct((NWORK, nc), F32),
        scratch_types=[
            pltpu.VMEM((nv,), F32),
            pltpu.VMEM((ept,), jnp.int32),
            pltpu.VMEM((ept,), jnp.int32),
            pltpu.VMEM((ept,), F32),
            pltpu.VMEM((nc,), F32),
        ],
    )
    def k(a_h, sv_h, dv_h, ef_h, o_h, av, svv, dvv, efvv, acc):
        w = _wid()
        base = w * ept
        pltpu.sync_copy(a_h, av)
        pltpu.sync_copy(sv_h.at[pl.ds(base, ept)], svv)
        pltpu.sync_copy(dv_h.at[pl.ds(base, ept)], dvv)
        pltpu.sync_copy(ef_h.at[pl.ds(base, ept)], efvv)
        zero16 = jnp.zeros((16,), F32)

        def zbody(i, _):
            acc[pl.ds(i * 16, 16)] = zero16
            return ()

        lax.fori_loop(0, nc // 16, zbody, ())
        lanes = lax.iota(jnp.int32, 16)

        def body(i, _):
            msk = lanes < (ept - i * 16)
            si = svv[pl.ds(i * 16, 16)]
            vals = plsc.load_gather(av, [si]) * efvv[pl.ds(i * 16, 16)]
            di = dvv[pl.ds(i * 16, 16)]
            plsc.addupdate_scatter(acc, [di], vals, mask=msk)
            return ()

        lax.fori_loop(0, niter, body, ())
        pltpu.sync_copy(acc, o_h.at[w])

    return k(a_vec, sv, dv, efv)


def kernel(var_node_features, con_node_features, edge_features_var,
           edge_features_con, rhs, obj, params, edge_index_var,
           edge_index_con, index, num_nodes_var, num_nodes_con):
    p = params
    nv = num_nodes_var.shape[0]
    nc = num_nodes_con.shape[0]
    E = edge_features_var.shape[0]
    e_count = float(E)

    sv = edge_index_var[0]
    dv = edge_index_var[1]
    sc = edge_index_con[0]
    dc = edge_index_con[1]
    efv_flat = edge_features_var.reshape(-1)
    index2d = index.reshape(nc, 1)

    def row(x):
        return x.reshape(1, -1)

    xv0, xc0 = tc_encode(var_node_features, con_node_features,
                         p["venc1_W"], row(p["venc1_b"]), p["venc2_W"],
                         row(p["venc2_b"]), p["cenc1_W"], row(p["cenc1_b"]),
                         p["cenc2_W"], row(p["cenc2_b"]))

    cntv_parts, cntc_parts = sc_counts(dv, dc, nc, nv)

    enc_ws = []
    for name in ("varE0", "varE1", "conE0", "conE1"):
        enc_ws.append((row(p[name + "_1_W"][0]), row(p[name + "_1_b"]),
                       p[name + "_2_W"], row(p[name + "_2_b"])))
    estats = tc_edge_stats(edge_features_var, edge_features_con, enc_ws)

    def enc_affine(k, name):
        mu = estats[2 * k] / e_count
        var = estats[2 * k + 1] / e_count - mu * mu
        s = p[name + "_bn_g"] * jax.lax.rsqrt(var + 1e-5)
        t = p[name + "_bn_be"] - mu * s
        return s, t

    xv, xc = xv0, xc0
    xvs = [xv0]
    for i in range(2):
        a, acon, bvar = tc_layer_prep(
            xv, xc, p[f"ass{i}_1_W"], row(p[f"ass{i}_1_b"]),
            p[f"ass{i}_2_W"], row(p[f"ass{i}_2_b"]),
            p[f"varN{i}_1_W"][0:H], p[f"varN{i}_1_W"][H:2 * H],
            row(p[f"varN{i}_1_W"][2 * H]))

        tmp_parts = sc_tmp(a.reshape(-1), sv, dv, efv_flat, nc, nv)
        err = tc_err(tmp_parts, rhs, index2d, row(p[f"err{i}_1_W"][0]),
                     row(p[f"err{i}_1_b"]), p[f"err{i}_2_W"],
                     row(p[f"err{i}_2_b"]), row(p[f"err{i}_bn_g"]),
                     row(p[f"err{i}_bn_be"]))

        se, te = enc_affine(i, f"varE{i}")
        w1d = p[f"varN{i}_1_W"][2 * H + 1:]
        wd = se[:, None] * w1d
        c0 = row(te @ w1d + p[f"varN{i}_1_b"])
        ag, bg = sc_gather_pair(acon, dv, bvar, sv)
        h2, st = tc_edge_main(ag, bg, edge_features_var,
                              enc_ws[i][0], enc_ws[i][1], enc_ws[i][2],
                              enc_ws[i][3], wd, c0, p[f"varN{i}_2_W"],
                              row(p[f"varN{i}_2_b"]))
        s0, s1 = sc_segsum(h2, dv, nc)
        w1x = p[f"conN{i}_1_W"]
        xc, pc, qv = tc_node_varN(s0, s1, cntv_parts, st,
                                  row(p[f"varN{i}_bn_g"]),
                                  row(p[f"varN{i}_bn_be"]), err, xv,
                                  w1x[0:H], w1x[H:2 * H], w1x[2 * H:3 * H],
                                  e_count)

        se2, te2 = enc_affine(2 + i, f"conE{i}")
        w1d2 = w1x[3 * H:]
        wd2 = se2[:, None] * w1d2
        c1 = row(te2 @ w1d2 + p[f"conN{i}_1_b"])
        qg, pg = sc_gather_pair(qv, dc, pc, sc)
        h2b, st2 = tc_edge_main(qg, pg, edge_features_con,
                                enc_ws[2 + i][0], enc_ws[2 + i][1],
                                enc_ws[2 + i][2], enc_ws[2 + i][3], wd2, c1,
                                p[f"conN{i}_2_W"], row(p[f"conN{i}_2_b"]))
        t0, t1 = sc_segsum(h2b, dc, nv)
        xv = tc_node_conN(t0, t1, cntc_parts, st2, row(p[f"conN{i}_bn_g"]),
                          row(p[f"conN{i}_bn_be"]), e_count)
        xvs.append(xv)

    out = tc_final(xvs[0], xvs[1], xvs[2], p["lin1_W"], row(p["lin1_b"]),
                   p["lin2_W"], row(p["lin2_b"]), p["lin3_W"],
                   row(p["lin3_b"]), p["lin4_W"], row(p["lin4_b"]))
    return out.reshape(-1)
